# Initial kernel scaffold; baseline (speedup 1.0000x reference)
import jax
import jax.numpy as jnp
from jax.experimental import pallas as pl


def kernel(x, edge_index, W1, b1, W2, b2, W3, b3, Wl1, bl1, Wl2, bl2, Wl3, bl3):
    raise NotImplementedError("write your pallas kernel here")


# speedup vs baseline: 9.2237x; 9.2237x over previous
import functools

import jax
import jax.numpy as jnp
from jax import lax
from jax.experimental import pallas as pl
from jax.experimental.pallas import tpu as pltpu
from jax.experimental.pallas import tpu_sc as plsc

N = 10000
E = 320000
NC = 2
NS = 16
NW = NC * NS
EPT = E // NW
B = 80
NP = 10240
RPT = NP // NS


def _sc_scatter_add(F):
  mesh = plsc.VectorSubcoreMesh(core_axis_name="c", subcore_axis_name="s")

  @functools.partial(
      pl.kernel,
      out_type=jax.ShapeDtypeStruct((NC * NP, F), jnp.float32),
      mesh=mesh,
      scratch_types=[
          pltpu.VMEM((B,), jnp.int32),
          pltpu.VMEM((B,), jnp.int32),
          pltpu.VMEM((B, F), jnp.float32),
          pltpu.VMEM_SHARED((NP, F), jnp.float32),
          pltpu.SemaphoreType.DMA,
      ],
  )
  def k(hp_hbm, src_hbm, dst_hbm, zeros_hbm, out_hbm, sidx, didx, rows, acc, sem):
    cid = lax.axis_index("c")
    sid = lax.axis_index("s")
    pltpu.sync_copy(zeros_hbm.at[pl.ds(sid * RPT, RPT)],
                    acc.at[pl.ds(sid * RPT, RPT)])
    plsc.subcore_barrier()
    base = (cid * NS + sid) * EPT

    def body(i, carry):
      off = base + i * B
      pltpu.sync_copy(src_hbm.at[pl.ds(off, B)], sidx)
      pltpu.sync_copy(dst_hbm.at[pl.ds(off, B)], didx)
      pltpu.async_copy(hp_hbm.at[sidx], rows, sem).wait()
      pltpu.sync_copy(rows, acc.at[didx], add=True)
      return carry

    lax.fori_loop(0, EPT // B, body, 0)
    plsc.subcore_barrier()
    pltpu.sync_copy(acc.at[pl.ds(sid * RPT, RPT)],
                    out_hbm.at[pl.ds(cid * NP + sid * RPT, RPT)])

  return k


def _sc_degree():
  mesh = plsc.VectorSubcoreMesh(core_axis_name="c", subcore_axis_name="s")

  @functools.partial(
      pl.kernel,
      out_type=jax.ShapeDtypeStruct((NC * NP, 1), jnp.float32),
      mesh=mesh,
      scratch_types=[
          pltpu.VMEM((B,), jnp.int32),
          pltpu.VMEM((B, 1), jnp.float32),
          pltpu.VMEM_SHARED((NP, 1), jnp.float32),
      ],
  )
  def k(dst_hbm, ones_hbm, zeros_hbm, out_hbm, didx, ones_v, acc):
    cid = lax.axis_index("c")
    sid = lax.axis_index("s")
    pltpu.sync_copy(ones_hbm, ones_v)
    pltpu.sync_copy(zeros_hbm.at[pl.ds(sid * RPT, RPT)],
                    acc.at[pl.ds(sid * RPT, RPT)])
    plsc.subcore_barrier()
    base = (cid * NS + sid) * EPT

    def body(i, carry):
      off = base + i * B
      pltpu.sync_copy(dst_hbm.at[pl.ds(off, B)], didx)
      pltpu.sync_copy(ones_v, acc.at[didx], add=True)
      return carry

    lax.fori_loop(0, EPT // B, body, 0)
    plsc.subcore_barrier()
    pltpu.sync_copy(acc.at[pl.ds(sid * RPT, RPT)],
                    out_hbm.at[pl.ds(cid * NP + sid * RPT, RPT)])

  return k


def _leaky(v):
  return jnp.where(v >= 0, v, 0.1 * v)


def _tc_prescale(deg_slabs, x, W1):
  def body(deg_ref, x_ref, w_ref, dinv_ref, hp_ref):
    deg = deg_ref[0:N, :] + deg_ref[NP:NP + N, :] + 1.0
    dinv = lax.rsqrt(deg)
    dinv_ref[...] = dinv
    hp_ref[...] = jnp.dot(x_ref[...], w_ref[...],
                          preferred_element_type=jnp.float32) * dinv

  return pl.pallas_call(
      body,
      out_shape=(jax.ShapeDtypeStruct((N, 1), jnp.float32),
                 jax.ShapeDtypeStruct((N, W1.shape[1]), jnp.float32)),
  )(deg_slabs, x, W1)


def _tc_fuse(slabs, hp, dinv, b, Wn):
  nslab = len(slabs)
  Fin = hp.shape[1]
  Fout = Wn.shape[1]

  def body(*refs):
    slab_refs = refs[:nslab]
    hp_ref, dinv_ref, b_ref, w_ref, out_ref = refs[nslab:]
    halves = [sr[0:N, :] + sr[NP:NP + N, :] for sr in slab_refs]
    S = halves[0] if nslab == 1 else jnp.concatenate(halves, axis=1)
    dinv = dinv_ref[...]
    act = _leaky(dinv * (S + hp_ref[...]) + b_ref[...])
    out_ref[...] = jnp.dot(act, w_ref[...],
                           preferred_element_type=jnp.float32) * dinv

  return pl.pallas_call(
      body,
      out_shape=jax.ShapeDtypeStruct((N, Fout), jnp.float32),
  )(*slabs, hp, dinv, b, Wn)


def _tc_act(slabs, hp, dinv, b):
  nslab = len(slabs)

  def body(*refs):
    slab_refs = refs[:nslab]
    hp_ref, dinv_ref, b_ref, out_ref = refs[nslab:]
    halves = [sr[0:N, :] + sr[NP:NP + N, :] for sr in slab_refs]
    S = halves[0] if nslab == 1 else jnp.concatenate(halves, axis=1)
    out_ref[...] = _leaky(dinv_ref[...] * (S + hp_ref[...]) + b_ref[...])

  return pl.pallas_call(
      body,
      out_shape=jax.ShapeDtypeStruct(hp.shape, jnp.float32),
  )(*slabs, hp, dinv, b)


def _tc_head(g, Wl1, bl1, Wl2, bl2, Wl3, bl3):
  def body(g_ref, w1_ref, b1_ref, w2_ref, b2_ref, w3_ref, b3_ref, out_ref):
    h = _leaky(jnp.dot(g_ref[...], w1_ref[...],
                       preferred_element_type=jnp.float32) + b1_ref[...])
    h = _leaky(jnp.dot(h, w2_ref[...],
                       preferred_element_type=jnp.float32) + b2_ref[...])
    h = jnp.dot(h, w3_ref[...], preferred_element_type=jnp.float32) + b3_ref[...]
    out_ref[...] = 1.0 / (1.0 + jnp.exp(-h))

  return pl.pallas_call(
      body,
      out_shape=jax.ShapeDtypeStruct((g.shape[0], Wl3.shape[1]), jnp.float32),
  )(g, Wl1, bl1, Wl2, bl2, Wl3, bl3)


def kernel(x, edge_index, W1, b1, W2, b2, W3, b3, Wl1, bl1, Wl2, bl2, Wl3, bl3):
  src = edge_index[0].astype(jnp.int32)
  dst = edge_index[1].astype(jnp.int32)
  zeros_w = jnp.zeros((NP, 128), jnp.float32)
  zeros_1 = jnp.zeros((NP, 1), jnp.float32)
  ones_b = jnp.ones((B, 1), jnp.float32)

  scat128 = _sc_scatter_add(128)
  degk = _sc_degree()

  deg_slabs = degk(dst, ones_b, zeros_1)
  dinv, hp1 = _tc_prescale(deg_slabs, x, W1)

  s1 = scat128(hp1, src, dst, zeros_w)
  hp2 = _tc_fuse([s1], hp1, dinv, b1.reshape(1, -1), W2)

  s2a = scat128(hp2[:, :128], src, dst, zeros_w)
  s2b = scat128(hp2[:, 128:], src, dst, zeros_w)
  hp3 = _tc_fuse([s2a, s2b], hp2, dinv, b2.reshape(1, -1), W3)

  s3 = scat128(hp3, src, dst, zeros_w)
  act3 = _tc_act([s3], hp3, dinv, b3.reshape(1, -1))

  g = act3.reshape(N // 40, 40 * 128)
  return _tc_head(g, Wl1, bl1.reshape(1, -1), Wl2, bl2.reshape(1, -1),
                  Wl3, bl3.reshape(1, -1))


# speedup vs baseline: 17.3653x; 1.8827x over previous
import functools

import jax
import jax.numpy as jnp
from jax import lax
from jax.experimental import pallas as pl
from jax.experimental.pallas import tpu as pltpu
from jax.experimental.pallas import tpu_sc as plsc

N = 10000
E = 320000
NC = 2
NS = 16
NW = NC * NS
EPT = E // NW
B = 80
NP = 10240
RPT = NP // NS


NCH = EPT // B


def _sc_scatter_add(F):
  mesh = plsc.VectorSubcoreMesh(core_axis_name="c", subcore_axis_name="s")

  @functools.partial(
      pl.kernel,
      out_type=jax.ShapeDtypeStruct((NC * NP, F), jnp.float32),
      mesh=mesh,
      scratch_types=[
          pltpu.VMEM((2, B), jnp.int32),
          pltpu.VMEM((2, B), jnp.int32),
          pltpu.VMEM((B, F), jnp.float32),
          pltpu.VMEM((B, F), jnp.float32),
          pltpu.VMEM_SHARED((NP, F), jnp.float32),
          pltpu.SemaphoreType.DMA,
          pltpu.SemaphoreType.DMA,
          pltpu.SemaphoreType.DMA,
          pltpu.SemaphoreType.DMA,
      ],
  )
  def k(hp_hbm, sd_hbm, zeros_hbm, out_hbm,
        sde, sdo, rowa, rowb, acc, ga, gb, sa, sb):
    cid = lax.axis_index("c")
    sid = lax.axis_index("s")
    wid = cid * NS + sid
    pltpu.sync_copy(zeros_hbm.at[pl.ds(sid * RPT, RPT)],
                    acc.at[pl.ds(sid * RPT, RPT)])
    plsc.subcore_barrier()

    def drain(buf, sem):
      pltpu.make_async_copy(hp_hbm.at[pl.ds(0, B)], buf, sem).wait()

    pltpu.sync_copy(sd_hbm.at[wid, 0], sde)
    pltpu.async_copy(hp_hbm.at[sde.at[0]], rowa, ga)

    def body(j, carry):
      i0 = 2 * j
      i1 = i0 + 1
      pltpu.sync_copy(sd_hbm.at[wid, i1], sdo)
      drain(rowa, ga)
      cpb = pltpu.async_copy(hp_hbm.at[sdo.at[0]], rowb, gb)
      csa = pltpu.async_copy(rowa, acc.at[sde.at[1]], sa, add=True)
      csa.wait()
      pltpu.sync_copy(sd_hbm.at[wid, i0 + 2], sde)
      cpb.wait()
      pltpu.async_copy(hp_hbm.at[sde.at[0]], rowa, ga)
      pltpu.async_copy(rowb, acc.at[sdo.at[1]], sb, add=True).wait()
      return carry

    lax.fori_loop(0, (NCH - 1) // 2, body, 0)
    drain(rowa, ga)
    pltpu.sync_copy(rowa, acc.at[sde.at[1]], add=True)
    plsc.subcore_barrier()
    pltpu.sync_copy(acc.at[pl.ds(sid * RPT, RPT)],
                    out_hbm.at[pl.ds(cid * NP + sid * RPT, RPT)])

  return k


def _sc_degree():
  mesh = plsc.VectorSubcoreMesh(core_axis_name="c", subcore_axis_name="s")

  @functools.partial(
      pl.kernel,
      out_type=jax.ShapeDtypeStruct((NC * NP, 1), jnp.float32),
      mesh=mesh,
      scratch_types=[
          pltpu.VMEM((NCH, B), jnp.int32),
          pltpu.VMEM((B, 1), jnp.float32),
          pltpu.VMEM_SHARED((NP, 1), jnp.float32),
          pltpu.SemaphoreType.DMA,
          pltpu.SemaphoreType.DMA,
      ],
  )
  def k(dst_hbm, ones_hbm, zeros_hbm, out_hbm, didx, ones_v, acc, sa, sb):
    cid = lax.axis_index("c")
    sid = lax.axis_index("s")
    wid = cid * NS + sid
    pltpu.sync_copy(dst_hbm.at[wid], didx)
    pltpu.sync_copy(ones_hbm, ones_v)
    pltpu.sync_copy(zeros_hbm.at[pl.ds(sid * RPT, RPT)],
                    acc.at[pl.ds(sid * RPT, RPT)])
    plsc.subcore_barrier()

    def body(j, carry):
      c0 = pltpu.async_copy(ones_v, acc.at[didx.at[2 * j]], sa, add=True)
      c1 = pltpu.async_copy(ones_v, acc.at[didx.at[2 * j + 1]], sb, add=True)
      c0.wait()
      c1.wait()
      return carry

    lax.fori_loop(0, NCH // 2, body, 0)
    pltpu.sync_copy(ones_v, acc.at[didx.at[NCH - 1]], add=True)
    plsc.subcore_barrier()
    pltpu.sync_copy(acc.at[pl.ds(sid * RPT, RPT)],
                    out_hbm.at[pl.ds(cid * NP + sid * RPT, RPT)])

  return k


def _leaky(v):
  return jnp.where(v >= 0, v, 0.1 * v)


def _tc_prescale(deg_slabs, x, W1):
  def body(deg_ref, x_ref, w_ref, dinv_ref, hp_ref):
    deg = deg_ref[0:N, :] + deg_ref[NP:NP + N, :] + 1.0
    dinv = lax.rsqrt(deg)
    dinv_ref[...] = dinv
    hp_ref[...] = jnp.dot(x_ref[...], w_ref[...],
                          preferred_element_type=jnp.float32) * dinv

  return pl.pallas_call(
      body,
      out_shape=(jax.ShapeDtypeStruct((N, 1), jnp.float32),
                 jax.ShapeDtypeStruct((N, W1.shape[1]), jnp.float32)),
  )(deg_slabs, x, W1)


def _tc_fuse(slabs, hp, dinv, b, Wn):
  nslab = len(slabs)
  Fin = hp.shape[1]
  Fout = Wn.shape[1]

  def body(*refs):
    slab_refs = refs[:nslab]
    hp_ref, dinv_ref, b_ref, w_ref, out_ref = refs[nslab:]
    halves = [sr[0:N, :] + sr[NP:NP + N, :] for sr in slab_refs]
    S = halves[0] if nslab == 1 else jnp.concatenate(halves, axis=1)
    dinv = dinv_ref[...]
    act = _leaky(dinv * (S + hp_ref[...]) + b_ref[...])
    out_ref[...] = jnp.dot(act, w_ref[...],
                           preferred_element_type=jnp.float32) * dinv

  return pl.pallas_call(
      body,
      out_shape=jax.ShapeDtypeStruct((N, Fout), jnp.float32),
  )(*slabs, hp, dinv, b, Wn)


def _tc_act(slabs, hp, dinv, b):
  nslab = len(slabs)

  def body(*refs):
    slab_refs = refs[:nslab]
    hp_ref, dinv_ref, b_ref, out_ref = refs[nslab:]
    halves = [sr[0:N, :] + sr[NP:NP + N, :] for sr in slab_refs]
    S = halves[0] if nslab == 1 else jnp.concatenate(halves, axis=1)
    out_ref[...] = _leaky(dinv_ref[...] * (S + hp_ref[...]) + b_ref[...])

  return pl.pallas_call(
      body,
      out_shape=jax.ShapeDtypeStruct(hp.shape, jnp.float32),
  )(*slabs, hp, dinv, b)


def _tc_head(g, Wl1, bl1, Wl2, bl2, Wl3, bl3):
  def body(g_ref, w1_ref, b1_ref, w2_ref, b2_ref, w3_ref, b3_ref, out_ref):
    h = _leaky(jnp.dot(g_ref[...], w1_ref[...],
                       preferred_element_type=jnp.float32) + b1_ref[...])
    h = _leaky(jnp.dot(h, w2_ref[...],
                       preferred_element_type=jnp.float32) + b2_ref[...])
    h = jnp.dot(h, w3_ref[...], preferred_element_type=jnp.float32) + b3_ref[...]
    out_ref[...] = 1.0 / (1.0 + jnp.exp(-h))

  return pl.pallas_call(
      body,
      out_shape=jax.ShapeDtypeStruct((g.shape[0], Wl3.shape[1]), jnp.float32),
  )(g, Wl1, bl1, Wl2, bl2, Wl3, bl3)


def kernel(x, edge_index, W1, b1, W2, b2, W3, b3, Wl1, bl1, Wl2, bl2, Wl3, bl3):
  src = edge_index[0].astype(jnp.int32).reshape(NW, NCH, B)
  dst = edge_index[1].astype(jnp.int32).reshape(NW, NCH, B)
  sd = jnp.stack([src, dst], axis=2)
  zeros_w = jnp.zeros((NP, 128), jnp.float32)
  zeros_1 = jnp.zeros((NP, 1), jnp.float32)
  ones_b = jnp.ones((B, 1), jnp.float32)

  scat128 = _sc_scatter_add(128)
  degk = _sc_degree()

  deg_slabs = degk(dst, ones_b, zeros_1)
  dinv, hp1 = _tc_prescale(deg_slabs, x, W1)

  s1 = scat128(hp1, sd, zeros_w)
  hp2 = _tc_fuse([s1], hp1, dinv, b1.reshape(1, -1), W2)

  s2a = scat128(hp2[:, :128], sd, zeros_w)
  s2b = scat128(hp2[:, 128:], sd, zeros_w)
  hp3 = _tc_fuse([s2a, s2b], hp2, dinv, b2.reshape(1, -1), W3)

  s3 = scat128(hp3, sd, zeros_w)
  act3 = _tc_act([s3], hp3, dinv, b3.reshape(1, -1))

  g = act3.reshape(N // 40, 40 * 128)
  return _tc_head(g, Wl1, bl1.reshape(1, -1), Wl2, bl2.reshape(1, -1),
                  Wl3, bl3.reshape(1, -1))
